# baseline (speedup 1.0000x reference)
import functools

import jax
import jax.numpy as jnp
from jax import lax
from jax.experimental import pallas as pl
from jax.experimental.pallas import tpu as pltpu
from jax.experimental.pallas import tpu_sc as plsc

_SC_PARAMS = pltpu.CompilerParams(use_tc_tiling_on_sc=False)

N = 10000
NP = 10240
E = 320000
CH = 128
NW = 32
NCHUNK = 80
EPW = NCHUNK * CH
EP = NW * EPW
NSUB = 16
RPS = NP // NSUB
NBUF = 2

D1 = 16
D2 = 48
NCLS = 40


def _sc_agg_kernel(D, K):
    KCH = K * CH
    NSUPER = NCHUNK // K
    mesh = plsc.VectorSubcoreMesh(core_axis_name="c", subcore_axis_name="s")

    @functools.partial(
        pl.kernel,
        out_type=jax.ShapeDtypeStruct((2, NP, D), jnp.float32),
        mesh=mesh,
        scratch_types=[
            pltpu.VMEM((EPW,), jnp.int32),
            pltpu.VMEM((EPW,), jnp.int32),
            [pltpu.VMEM((KCH, D), jnp.float32) for _ in range(NBUF)],
            pltpu.VMEM_SHARED((NP, D), jnp.float32),
            [pltpu.SemaphoreType.DMA for _ in range(NBUF)],
        ],
        compiler_params=_SC_PARAMS,
    )
    def kernel(y_hbm, src_hbm, dst_hbm, out_hbm, sidx, didx, rows, acc, sems):
        cid = lax.axis_index("c")
        sid = lax.axis_index("s")
        wid = sid * 2 + cid
        pltpu.sync_copy(src_hbm.at[wid], sidx)
        pltpu.sync_copy(dst_hbm.at[wid], didx)

        @pl.loop(0, RPS)
        def _(r):
            @pl.loop(0, D, step=16)
            def _(c2):
                rows[0][r, pl.ds(c2, 16)] = jnp.zeros((16,), jnp.float32)

        pltpu.sync_copy(rows[0].at[pl.ds(0, RPS)],
                        acc.at[pl.ds(sid * RPS, RPS)])

        plsc.subcore_barrier()

        for b in range(NBUF):
            pltpu.async_copy(
                y_hbm.at[sidx.at[pl.ds(b * KCH, KCH)]], rows[b], sems[b])

        @pl.loop(0, NSUPER - NBUF, step=NBUF)
        def _(i):
            for b in range(NBUF):
                pltpu.make_async_copy(
                    y_hbm.at[sidx.at[pl.ds((i + b) * KCH, KCH)]],
                    rows[b], sems[b]).wait()
                pltpu.sync_copy(
                    rows[b], acc.at[didx.at[pl.ds((i + b) * KCH, KCH)]],
                    add=True)
                pltpu.async_copy(
                    y_hbm.at[sidx.at[pl.ds((i + b + NBUF) * KCH, KCH)]],
                    rows[b], sems[b])

        for b in range(NBUF):
            c = NSUPER - NBUF + b
            pltpu.make_async_copy(
                y_hbm.at[sidx.at[pl.ds(c * KCH, KCH)]], rows[b], sems[b]).wait()
            pltpu.sync_copy(
                rows[b], acc.at[didx.at[pl.ds(c * KCH, KCH)]], add=True)

        plsc.subcore_barrier()
        pltpu.sync_copy(
            acc.at[pl.ds(sid * RPS, RPS)],
            out_hbm.at[cid, pl.ds(sid * RPS, RPS)],
        )

    return kernel


def _sc_deg_kernel():
    K = 8
    KCH = K * CH
    NSUPER = NCHUNK // K
    mesh = plsc.VectorSubcoreMesh(core_axis_name="c", subcore_axis_name="s")

    @functools.partial(
        pl.kernel,
        out_type=jax.ShapeDtypeStruct((2, NP, 16), jnp.float32),
        mesh=mesh,
        scratch_types=[
            pltpu.VMEM((EPW,), jnp.int32),
            pltpu.VMEM((KCH, 16), jnp.float32),
            pltpu.VMEM_SHARED((NP, 16), jnp.float32),
            pltpu.SemaphoreType.DMA,
        ],
        compiler_params=_SC_PARAMS,
    )
    def kernel(dst_hbm, out_hbm, didx, vals, acc, sem):
        cid = lax.axis_index("c")
        sid = lax.axis_index("s")
        wid = sid * 2 + cid
        pltpu.sync_copy(dst_hbm.at[wid], didx)

        @pl.loop(0, RPS)
        def _(r):
            vals[r, pl.ds(0, 16)] = jnp.zeros((16,), jnp.float32)

        pltpu.sync_copy(vals.at[pl.ds(0, RPS)], acc.at[pl.ds(sid * RPS, RPS)])

        @pl.loop(0, KCH)
        def _(r):
            vals[r, pl.ds(0, 16)] = jnp.full((16,), 1.0, jnp.float32)

        plsc.subcore_barrier()

        @pl.loop(0, NSUPER)
        def _(i):
            pltpu.sync_copy(vals, acc.at[didx.at[pl.ds(i * KCH, KCH)]],
                            add=True)

        plsc.subcore_barrier()
        pltpu.sync_copy(
            acc.at[pl.ds(sid * RPS, RPS)],
            out_hbm.at[cid, pl.ds(sid * RPS, RPS)],
        )

    return kernel


def _tc_layer1(x_pad, W1, degacc):
    def body(x_ref, w_ref, d_ref, y1_ref, dinv_ref):
        d = d_ref[...]
        deg = d[0, :, 0:1] + d[1, :, 0:1] + 1.0
        dinv = lax.rsqrt(deg)
        xw = jnp.dot(x_ref[...], w_ref[...], preferred_element_type=jnp.float32)
        y1_ref[...] = xw * dinv
        dinv_ref[...] = dinv

    return pl.pallas_call(
        body,
        out_shape=(
            jax.ShapeDtypeStruct((NP, D1), jnp.float32),
            jax.ShapeDtypeStruct((NP, 1), jnp.float32),
        ),
    )(x_pad, W1, degacc)


def _tc_layer2(agg1, y1, dinv, b1, W2p):
    def body(a_ref, y1_ref, dinv_ref, b1_ref, w2_ref, y2_ref):
        a = a_ref[...]
        dinv = dinv_ref[...]
        s = (a[0] + a[1] + y1_ref[...]) * dinv + b1_ref[...]
        h = jnp.maximum(s, 0.0)
        y2 = jnp.dot(h, w2_ref[...], preferred_element_type=jnp.float32) * dinv
        row = lax.broadcasted_iota(jnp.int32, (NP, D2), 0)
        y2_ref[...] = jnp.where(row < N, y2, 0.0)

    return pl.pallas_call(
        body,
        out_shape=jax.ShapeDtypeStruct((NP, D2), jnp.float32),
    )(agg1, y1, dinv, b1, W2p)


def _tc_final(agg2, y2, dinv, b2):
    def body(a_ref, y2_ref, dinv_ref, b2_ref, o_ref):
        a = a_ref[...]
        s = (a[0] + a[1] + y2_ref[...]) * dinv_ref[...]
        o = s[:N, :NCLS] + b2_ref[...]
        m = jnp.max(o, axis=1, keepdims=True)
        e = jnp.exp(o - m)
        lse = jnp.log(jnp.sum(e, axis=1, keepdims=True))
        o_ref[...] = o - m - lse

    return pl.pallas_call(
        body,
        out_shape=jax.ShapeDtypeStruct((N, NCLS), jnp.float32),
    )(agg2, y2, dinv, b2)


def kernel(x, edge_index, W1, b1, W2, b2):
    src = edge_index[0].astype(jnp.int32)
    dst = edge_index[1].astype(jnp.int32)
    npad = EP - E
    src_p = jnp.concatenate([src, jnp.full((npad,), N, jnp.int32)])
    dst_p = jnp.concatenate(
        [dst, N + (jnp.arange(npad, dtype=jnp.int32) % (NP - N))]
    )
    src_p = src_p.reshape(NW, EPW)
    dst_p = dst_p.reshape(NW, EPW)
    x_pad = jnp.pad(x, ((0, NP - N), (0, 0)))
    b1r = b1.reshape(1, D1)
    W2p = jnp.pad(W2, ((0, 0), (0, D2 - NCLS)))
    b2r = b2.reshape(1, NCLS)

    degacc = _sc_deg_kernel()(dst_p)
    y1, dinv = _tc_layer1(x_pad, W1, degacc)
    agg1 = _sc_agg_kernel(D1, 8)(y1, src_p, dst_p)
    y2 = _tc_layer2(agg1, y1, dinv, b1r, W2p)
    agg2 = _sc_agg_kernel(D2, 4)(y2, src_p, dst_p)
    return _tc_final(agg2, y2, dinv, b2r)


# speedup vs baseline: 1.8111x; 1.8111x over previous
import functools

import jax
import jax.numpy as jnp
from jax import lax
from jax.experimental import pallas as pl
from jax.experimental.pallas import tpu as pltpu
from jax.experimental.pallas import tpu_sc as plsc

_SC_PARAMS = pltpu.CompilerParams(use_tc_tiling_on_sc=False)

N = 10000
NP = 10240
E = 320000
CH = 128
NW = 32
NCHUNK = 80
EPW = NCHUNK * CH
EP = NW * EPW
NSUB = 16
RPS = NP // NSUB
NBUF = 2

D1 = 16
D2 = 48
NCLS = 40


def _sc_agg_kernel(D, K):
    KCH = K * CH
    NSUPER = NCHUNK // K
    mesh = plsc.VectorSubcoreMesh(core_axis_name="c", subcore_axis_name="s")

    @functools.partial(
        pl.kernel,
        out_type=jax.ShapeDtypeStruct((2, NP, D), jnp.float32),
        mesh=mesh,
        scratch_types=[
            pltpu.VMEM((EPW,), jnp.int32),
            pltpu.VMEM((EPW,), jnp.int32),
            [pltpu.VMEM((KCH, D), jnp.float32) for _ in range(NBUF)],
            pltpu.VMEM_SHARED((NP, D), jnp.float32),
            [pltpu.SemaphoreType.DMA for _ in range(NBUF)],
        ],
        compiler_params=_SC_PARAMS,
    )
    def kernel(y_hbm, src_hbm, dst_hbm, out_hbm, sidx, didx, rows, acc, sems):
        cid = lax.axis_index("c")
        sid = lax.axis_index("s")
        wid = sid * 2 + cid
        pltpu.sync_copy(src_hbm.at[wid], sidx)
        pltpu.sync_copy(dst_hbm.at[wid], didx)

        @pl.loop(0, RPS)
        def _(r):
            @pl.loop(0, D, step=16)
            def _(c2):
                rows[0][r, pl.ds(c2, 16)] = jnp.zeros((16,), jnp.float32)

        pltpu.sync_copy(rows[0].at[pl.ds(0, RPS)],
                        acc.at[pl.ds(sid * RPS, RPS)])

        plsc.subcore_barrier()

        for b in range(NBUF):
            pltpu.async_copy(
                y_hbm.at[sidx.at[pl.ds(b * KCH, KCH)]], rows[b], sems[b])

        @pl.loop(0, NSUPER - NBUF, step=NBUF)
        def _(i):
            for b in range(NBUF):
                pltpu.make_async_copy(
                    y_hbm.at[sidx.at[pl.ds((i + b) * KCH, KCH)]],
                    rows[b], sems[b]).wait()
                pltpu.sync_copy(
                    rows[b], acc.at[didx.at[pl.ds((i + b) * KCH, KCH)]],
                    add=True)
                pltpu.async_copy(
                    y_hbm.at[sidx.at[pl.ds((i + b + NBUF) * KCH, KCH)]],
                    rows[b], sems[b])

        for b in range(NBUF):
            c = NSUPER - NBUF + b
            pltpu.make_async_copy(
                y_hbm.at[sidx.at[pl.ds(c * KCH, KCH)]], rows[b], sems[b]).wait()
            pltpu.sync_copy(
                rows[b], acc.at[didx.at[pl.ds(c * KCH, KCH)]], add=True)

        plsc.subcore_barrier()
        pltpu.sync_copy(
            acc.at[pl.ds(sid * RPS, RPS)],
            out_hbm.at[cid, pl.ds(sid * RPS, RPS)],
        )

    return kernel


def _sc_deg_kernel():
    K = 8
    KCH = K * CH
    NSUPER = NCHUNK // K
    mesh = plsc.VectorSubcoreMesh(core_axis_name="c", subcore_axis_name="s")

    @functools.partial(
        pl.kernel,
        out_type=jax.ShapeDtypeStruct((2, NP, 16), jnp.float32),
        mesh=mesh,
        scratch_types=[
            pltpu.VMEM((EPW,), jnp.int32),
            pltpu.VMEM((KCH, 16), jnp.float32),
            pltpu.VMEM_SHARED((NP, 16), jnp.float32),
            pltpu.SemaphoreType.DMA,
        ],
        compiler_params=_SC_PARAMS,
    )
    def kernel(dst_hbm, out_hbm, didx, vals, acc, sem):
        cid = lax.axis_index("c")
        sid = lax.axis_index("s")
        wid = sid * 2 + cid
        pltpu.sync_copy(dst_hbm.at[wid], didx)

        @pl.loop(0, RPS)
        def _(r):
            vals[r, pl.ds(0, 16)] = jnp.zeros((16,), jnp.float32)

        pltpu.sync_copy(vals.at[pl.ds(0, RPS)], acc.at[pl.ds(sid * RPS, RPS)])

        @pl.loop(0, KCH)
        def _(r):
            vals[r, pl.ds(0, 16)] = jnp.full((16,), 1.0, jnp.float32)

        plsc.subcore_barrier()

        @pl.loop(0, NSUPER)
        def _(i):
            pltpu.sync_copy(vals, acc.at[didx.at[pl.ds(i * KCH, KCH)]],
                            add=True)

        plsc.subcore_barrier()
        pltpu.sync_copy(
            acc.at[pl.ds(sid * RPS, RPS)],
            out_hbm.at[cid, pl.ds(sid * RPS, RPS)],
        )

    return kernel


def _tc_layer1(x_pad, W1, degacc):
    def body(x_ref, w_ref, d_ref, y1_ref, dinv_ref):
        d = d_ref[...]
        deg = d[0, :, 0:1] + d[1, :, 0:1] + 1.0
        dinv = lax.rsqrt(deg)
        xw = jnp.dot(x_ref[...], w_ref[...], preferred_element_type=jnp.float32)
        y1_ref[...] = xw * dinv
        dinv_ref[...] = dinv

    return pl.pallas_call(
        body,
        out_shape=(
            jax.ShapeDtypeStruct((NP, D1), jnp.float32),
            jax.ShapeDtypeStruct((NP, 1), jnp.float32),
        ),
    )(x_pad, W1, degacc)


def _tc_layer2(agg1, y1, dinv, b1, W2p):
    def body(a_ref, y1_ref, dinv_ref, b1_ref, w2_ref, y2_ref):
        a = a_ref[...]
        dinv = dinv_ref[...]
        s = (a[0] + a[1] + y1_ref[...]) * dinv + b1_ref[...]
        h = jnp.maximum(s, 0.0)
        y2 = jnp.dot(h, w2_ref[...], preferred_element_type=jnp.float32) * dinv
        row = lax.broadcasted_iota(jnp.int32, (NP, D2), 0)
        y2_ref[...] = jnp.where(row < N, y2, 0.0)

    return pl.pallas_call(
        body,
        out_shape=jax.ShapeDtypeStruct((NP, D2), jnp.float32),
    )(agg1, y1, dinv, b1, W2p)


def _tc_final(agg2, y2, dinv, b2):
    def body(a_ref, y2_ref, dinv_ref, b2_ref, o_ref):
        a = a_ref[...]
        s = (a[0] + a[1] + y2_ref[...]) * dinv_ref[...]
        o = s[:N, :NCLS] + b2_ref[...]
        m = jnp.max(o, axis=1, keepdims=True)
        e = jnp.exp(o - m)
        lse = jnp.log(jnp.sum(e, axis=1, keepdims=True))
        o_ref[...] = o - m - lse

    return pl.pallas_call(
        body,
        out_shape=jax.ShapeDtypeStruct((N, NCLS), jnp.float32),
    )(agg2, y2, dinv, b2)


def kernel(x, edge_index, W1, b1, W2, b2):
    src = edge_index[0].astype(jnp.int32)
    dst = edge_index[1].astype(jnp.int32)
    npad = EP - E
    pad_iota = N + (jnp.arange(npad, dtype=jnp.int32) % (NP - N))
    src_p = jnp.concatenate([src, pad_iota])
    dst_p = jnp.concatenate([dst, pad_iota])
    src_p = src_p.reshape(NW, EPW)
    dst_p = dst_p.reshape(NW, EPW)
    x_pad = jnp.pad(x, ((0, NP - N), (0, 0)))
    b1r = b1.reshape(1, D1)
    W2p = jnp.pad(W2, ((0, 0), (0, D2 - NCLS)))
    b2r = b2.reshape(1, NCLS)

    degacc = _sc_deg_kernel()(dst_p)
    y1, dinv = _tc_layer1(x_pad, W1, degacc)
    agg1 = _sc_agg_kernel(D1, 8)(y1, src_p, dst_p)
    y2 = _tc_layer2(agg1, y1, dinv, b1r, W2p)
    agg2 = _sc_agg_kernel(D2, 4)(y2, src_p, dst_p)
    return _tc_final(agg2, y2, dinv, b2r)


# speedup vs baseline: 2.0102x; 1.1099x over previous
import functools

import jax
import jax.numpy as jnp
from jax import lax
from jax.experimental import pallas as pl
from jax.experimental.pallas import tpu as pltpu
from jax.experimental.pallas import tpu_sc as plsc

_SC_PARAMS = pltpu.CompilerParams(use_tc_tiling_on_sc=False)

N = 10000
NP = 10240
E = 320000
NW = 32
EPW = E // NW
NSUB = 16
RPS = NP // NSUB
NBUF = 2

D1 = 16
D2 = 64
NCLS = 40


def _sc_agg_kernel(D, KCH):
    NSUPER = EPW // KCH
    ZR = 640 if KCH >= 640 else 160
    mesh = plsc.VectorSubcoreMesh(core_axis_name="c", subcore_axis_name="s")

    @functools.partial(
        pl.kernel,
        out_type=jax.ShapeDtypeStruct((2, NP, 128), jnp.float32),
        mesh=mesh,
        scratch_types=[
            pltpu.VMEM((EPW,), jnp.int32),
            pltpu.VMEM((EPW,), jnp.int32),
            [pltpu.VMEM((KCH, D), jnp.float32) for _ in range(NBUF)],
            pltpu.VMEM_SHARED((NP, D), jnp.float32),
            [pltpu.SemaphoreType.DMA for _ in range(NBUF)],
        ],
        compiler_params=_SC_PARAMS,
    )
    def kernel(y_hbm, src_hbm, dst_hbm, out_hbm, sidx, didx, rows, acc, sems):
        cid = lax.axis_index("c")
        sid = lax.axis_index("s")
        wid = sid * 2 + cid
        pltpu.sync_copy(src_hbm.at[pl.ds(wid * EPW, EPW)], sidx)
        pltpu.sync_copy(dst_hbm.at[pl.ds(wid * EPW, EPW)], didx)

        @pl.loop(0, ZR)
        def _(r):
            @pl.loop(0, D, step=16)
            def _(c2):
                rows[0][r, pl.ds(c2, 16)] = jnp.zeros((16,), jnp.float32)

        for j in range(RPS // ZR):
            pltpu.sync_copy(rows[0].at[pl.ds(0, ZR)],
                            acc.at[pl.ds(sid * RPS + j * ZR, ZR)])

        plsc.subcore_barrier()

        for b in range(NBUF):
            pltpu.async_copy(
                y_hbm.at[sidx.at[pl.ds(b * KCH, KCH)]], rows[b], sems[b])

        @pl.loop(0, NSUPER - NBUF, step=NBUF)
        def _(i):
            for b in range(NBUF):
                pltpu.make_async_copy(
                    y_hbm.at[sidx.at[pl.ds((i + b) * KCH, KCH)]],
                    rows[b], sems[b]).wait()
                pltpu.sync_copy(
                    rows[b], acc.at[didx.at[pl.ds((i + b) * KCH, KCH)]],
                    add=True)
                pltpu.async_copy(
                    y_hbm.at[sidx.at[pl.ds((i + b + NBUF) * KCH, KCH)]],
                    rows[b], sems[b])

        for b in range(NBUF):
            c = NSUPER - NBUF + b
            pltpu.make_async_copy(
                y_hbm.at[sidx.at[pl.ds(c * KCH, KCH)]], rows[b], sems[b]).wait()
            pltpu.sync_copy(
                rows[b], acc.at[didx.at[pl.ds(c * KCH, KCH)]], add=True)

        plsc.subcore_barrier()
        pltpu.sync_copy(
            acc.at[pl.ds(sid * RPS, RPS)],
            out_hbm.at[cid, pl.ds(sid * RPS, RPS), pl.ds(0, D)],
        )

    return kernel


def _sc_deg_kernel():
    KCH = 1000
    NSUPER = EPW // KCH
    mesh = plsc.VectorSubcoreMesh(core_axis_name="c", subcore_axis_name="s")

    @functools.partial(
        pl.kernel,
        out_type=jax.ShapeDtypeStruct((2, NP, 128), jnp.float32),
        mesh=mesh,
        scratch_types=[
            pltpu.VMEM((EPW,), jnp.int32),
            pltpu.VMEM((KCH, 16), jnp.float32),
            pltpu.VMEM_SHARED((NP, 16), jnp.float32),
            pltpu.SemaphoreType.DMA,
        ],
        compiler_params=_SC_PARAMS,
    )
    def kernel(dst_hbm, out_hbm, didx, vals, acc, sem):
        cid = lax.axis_index("c")
        sid = lax.axis_index("s")
        wid = sid * 2 + cid
        pltpu.sync_copy(dst_hbm.at[pl.ds(wid * EPW, EPW)], didx)

        @pl.loop(0, RPS)
        def _(r):
            vals[r, pl.ds(0, 16)] = jnp.zeros((16,), jnp.float32)

        pltpu.sync_copy(vals.at[pl.ds(0, RPS)], acc.at[pl.ds(sid * RPS, RPS)])

        @pl.loop(0, KCH)
        def _(r):
            vals[r, pl.ds(0, 16)] = jnp.full((16,), 1.0, jnp.float32)

        plsc.subcore_barrier()

        @pl.loop(0, NSUPER)
        def _(i):
            pltpu.sync_copy(vals, acc.at[didx.at[pl.ds(i * KCH, KCH)]],
                            add=True)

        plsc.subcore_barrier()
        pltpu.sync_copy(
            acc.at[pl.ds(sid * RPS, RPS)],
            out_hbm.at[cid, pl.ds(sid * RPS, RPS), pl.ds(0, 16)],
        )

    return kernel


def _tc_layer1(x_pad, W1, degacc):
    def body(x_ref, w_ref, d_ref, y1_ref, dinv_ref):
        d = d_ref[...]
        dinv16 = lax.rsqrt(d[0, :, :D1] + d[1, :, :D1] + 1.0)
        xw = jnp.dot(x_ref[...], w_ref[...], preferred_element_type=jnp.float32)
        y1_ref[:, :D1] = xw * dinv16
        dinv_ref[...] = dinv16

    return pl.pallas_call(
        body,
        out_shape=(
            jax.ShapeDtypeStruct((NP, 128), jnp.float32),
            jax.ShapeDtypeStruct((NP, D1), jnp.float32),
        ),
    )(x_pad, W1, degacc)


def _tc_layer2(agg1, y1f, dinv16, b1, W2p):
    def body(a_ref, y1_ref, dinv_ref, b1_ref, w2_ref, y2_ref):
        a = a_ref[...]
        dinv16 = dinv_ref[...]
        s = (a[0, :, :D1] + a[1, :, :D1] + y1_ref[:, :D1]) * dinv16
        h = jnp.maximum(s + b1_ref[...], 0.0)
        y2 = jnp.dot(h, w2_ref[...], preferred_element_type=jnp.float32)
        y2_ref[:, :D2] = y2 * dinv16[:, 0:1]

    return pl.pallas_call(
        body,
        out_shape=jax.ShapeDtypeStruct((NP, 128), jnp.float32),
    )(agg1, y1f, dinv16, b1, W2p)


def _tc_final(agg2, y2f, dinv16, b2):
    def body(a_ref, y2_ref, dinv_ref, b2_ref, o_ref):
        a = a_ref[...]
        t = a[0, :N, :NCLS] + a[1, :N, :NCLS] + y2_ref[:N, :NCLS]
        o = t * dinv_ref[:N, 0:1] + b2_ref[...]
        m = jnp.max(o, axis=1, keepdims=True)
        e = jnp.exp(o - m)
        lse = jnp.log(jnp.sum(e, axis=1, keepdims=True))
        o_ref[...] = o - m - lse

    return pl.pallas_call(
        body,
        out_shape=jax.ShapeDtypeStruct((N, NCLS), jnp.float32),
    )(agg2, y2f, dinv16, b2)


def kernel(x, edge_index, W1, b1, W2, b2):
    edges = edge_index.astype(jnp.int32)
    src = edges[0]
    dst = edges[1]
    src8 = src * 8
    src2 = src * 2
    x_pad = jnp.pad(x, ((0, NP - N), (0, 0)))
    b1r = b1.reshape(1, D1)
    W2p = jnp.pad(W2, ((0, 0), (0, D2 - NCLS)))
    b2r = b2.reshape(1, NCLS)

    degacc = _sc_deg_kernel()(dst)
    y1f, dinv16 = _tc_layer1(x_pad, W1, degacc)
    agg1 = _sc_agg_kernel(D1, 1000)(y1f.reshape(8 * NP, 16), src8, dst)
    y2f = _tc_layer2(agg1, y1f, dinv16, b1r, W2p)
    agg2 = _sc_agg_kernel(D2, 200)(y2f.reshape(2 * NP, 64), src2, dst)
    return _tc_final(agg2, y2f, dinv16, b2r)


# speedup vs baseline: 2.0558x; 1.0227x over previous
import functools

import jax
import jax.numpy as jnp
from jax import lax
from jax.experimental import pallas as pl
from jax.experimental.pallas import tpu as pltpu
from jax.experimental.pallas import tpu_sc as plsc

_SC_PARAMS = pltpu.CompilerParams(use_tc_tiling_on_sc=False)

N = 10000
NP = 10240
E = 320000
NW = 32
EPW = E // NW
NSUB = 16
RPS = NP // NSUB
NBUF = 2

D1 = 16
D2 = 64
NCLS = 40


def _sc_agg_kernel(D, KCH, SHIFT):
    NSUPER = EPW // KCH
    MAIN = NBUF * ((NSUPER - NBUF) // NBUF)
    ZR = 640 if KCH >= 640 else 160
    mesh = plsc.VectorSubcoreMesh(core_axis_name="c", subcore_axis_name="s")

    @functools.partial(
        pl.kernel,
        out_type=jax.ShapeDtypeStruct((2, NP, 128), jnp.float32),
        mesh=mesh,
        scratch_types=[
            pltpu.VMEM((EPW,), jnp.int32),
            pltpu.VMEM((EPW,), jnp.int32),
            [pltpu.VMEM((KCH, D), jnp.float32) for _ in range(NBUF)],
            pltpu.VMEM_SHARED((NP, D), jnp.float32),
            [pltpu.SemaphoreType.DMA for _ in range(NBUF)],
        ],
        compiler_params=_SC_PARAMS,
    )
    def kernel(y_hbm, edge_hbm, out_hbm, sidx, didx, rows, acc, sems):
        cid = lax.axis_index("c")
        sid = lax.axis_index("s")
        wid = sid * 2 + cid
        pltpu.sync_copy(edge_hbm.at[0, pl.ds(wid * EPW, EPW)], sidx)
        pltpu.sync_copy(edge_hbm.at[1, pl.ds(wid * EPW, EPW)], didx)

        @pl.loop(0, EPW, step=16)
        def _(r):
            sidx[pl.ds(r, 16)] = sidx[pl.ds(r, 16)] * (1 << SHIFT)

        @pl.loop(0, ZR)
        def _(r):
            @pl.loop(0, D, step=16)
            def _(c2):
                rows[0][r, pl.ds(c2, 16)] = jnp.zeros((16,), jnp.float32)

        for j in range(RPS // ZR):
            pltpu.sync_copy(rows[0].at[pl.ds(0, ZR)],
                            acc.at[pl.ds(sid * RPS + j * ZR, ZR)])

        plsc.subcore_barrier()

        for b in range(NBUF):
            pltpu.async_copy(
                y_hbm.at[sidx.at[pl.ds(b * KCH, KCH)]], rows[b], sems[b])

        @pl.loop(0, MAIN, step=NBUF)
        def _(i):
            for b in range(NBUF):
                pltpu.make_async_copy(
                    y_hbm.at[sidx.at[pl.ds((i + b) * KCH, KCH)]],
                    rows[b], sems[b]).wait()
                pltpu.sync_copy(
                    rows[b], acc.at[didx.at[pl.ds((i + b) * KCH, KCH)]],
                    add=True)
                pltpu.async_copy(
                    y_hbm.at[sidx.at[pl.ds((i + b + NBUF) * KCH, KCH)]],
                    rows[b], sems[b])

        for c in range(MAIN, NSUPER):
            b = c % NBUF
            pltpu.make_async_copy(
                y_hbm.at[sidx.at[pl.ds(c * KCH, KCH)]], rows[b], sems[b]).wait()
            pltpu.sync_copy(
                rows[b], acc.at[didx.at[pl.ds(c * KCH, KCH)]], add=True)
            if c + NBUF < NSUPER:
                pltpu.async_copy(
                    y_hbm.at[sidx.at[pl.ds((c + NBUF) * KCH, KCH)]],
                    rows[b], sems[b])

        plsc.subcore_barrier()
        pltpu.sync_copy(
            acc.at[pl.ds(sid * RPS, RPS)],
            out_hbm.at[cid, pl.ds(sid * RPS, RPS), pl.ds(0, D)],
        )

    return kernel


def _sc_deg_kernel():
    KCH = 2000
    NSUPER = EPW // KCH
    mesh = plsc.VectorSubcoreMesh(core_axis_name="c", subcore_axis_name="s")

    @functools.partial(
        pl.kernel,
        out_type=jax.ShapeDtypeStruct((2, NP, 128), jnp.float32),
        mesh=mesh,
        scratch_types=[
            pltpu.VMEM((EPW,), jnp.int32),
            pltpu.VMEM((KCH, 16), jnp.float32),
            pltpu.VMEM_SHARED((NP, 16), jnp.float32),
            pltpu.SemaphoreType.DMA,
        ],
        compiler_params=_SC_PARAMS,
    )
    def kernel(edge_hbm, out_hbm, didx, vals, acc, sem):
        cid = lax.axis_index("c")
        sid = lax.axis_index("s")
        wid = sid * 2 + cid
        pltpu.sync_copy(edge_hbm.at[1, pl.ds(wid * EPW, EPW)], didx)

        @pl.loop(0, RPS)
        def _(r):
            vals[r, pl.ds(0, 16)] = jnp.zeros((16,), jnp.float32)

        pltpu.sync_copy(vals.at[pl.ds(0, RPS)], acc.at[pl.ds(sid * RPS, RPS)])

        @pl.loop(0, KCH)
        def _(r):
            vals[r, pl.ds(0, 16)] = jnp.full((16,), 1.0, jnp.float32)

        plsc.subcore_barrier()

        @pl.loop(0, NSUPER)
        def _(i):
            pltpu.sync_copy(vals, acc.at[didx.at[pl.ds(i * KCH, KCH)]],
                            add=True)

        plsc.subcore_barrier()
        pltpu.sync_copy(
            acc.at[pl.ds(sid * RPS, RPS)],
            out_hbm.at[cid, pl.ds(sid * RPS, RPS), pl.ds(0, 16)],
        )

    return kernel


def _tc_layer1(x_pad, W1, degacc):
    def body(x_ref, w_ref, d_ref, y1_ref, dinv_ref):
        d = d_ref[...]
        dinv16 = lax.rsqrt(d[0, :, :D1] + d[1, :, :D1] + 1.0)
        xw = jnp.dot(x_ref[...], w_ref[...], preferred_element_type=jnp.float32)
        y1_ref[:, :D1] = xw * dinv16
        dinv_ref[...] = dinv16

    return pl.pallas_call(
        body,
        out_shape=(
            jax.ShapeDtypeStruct((NP, 128), jnp.float32),
            jax.ShapeDtypeStruct((NP, D1), jnp.float32),
        ),
    )(x_pad, W1, degacc)


def _tc_layer2(agg1, y1f, dinv16, b1, W2p):
    def body(a_ref, y1_ref, dinv_ref, b1_ref, w2_ref, y2_ref):
        a = a_ref[...]
        dinv16 = dinv_ref[...]
        s = (a[0, :, :D1] + a[1, :, :D1] + y1_ref[:, :D1]) * dinv16
        h = jnp.maximum(s + b1_ref[...], 0.0)
        y2 = jnp.dot(h, w2_ref[...], preferred_element_type=jnp.float32)
        y2_ref[:, :D2] = y2 * dinv16[:, 0:1]

    return pl.pallas_call(
        body,
        out_shape=jax.ShapeDtypeStruct((NP, 128), jnp.float32),
    )(agg1, y1f, dinv16, b1, W2p)


def _tc_final(agg2, y2f, dinv16, b2):
    def body(a_ref, y2_ref, dinv_ref, b2_ref, o_ref):
        a = a_ref[...]
        t = a[0, :N, :NCLS] + a[1, :N, :NCLS] + y2_ref[:N, :NCLS]
        o = t * dinv_ref[:N, 0:1] + b2_ref[...]
        m = jnp.max(o, axis=1, keepdims=True)
        e = jnp.exp(o - m)
        lse = jnp.log(jnp.sum(e, axis=1, keepdims=True))
        o_ref[...] = o - m - lse

    return pl.pallas_call(
        body,
        out_shape=jax.ShapeDtypeStruct((N, NCLS), jnp.float32),
    )(agg2, y2f, dinv16, b2)


def kernel(x, edge_index, W1, b1, W2, b2):
    edges = edge_index.astype(jnp.int32)
    x_pad = jnp.pad(x, ((0, NP - N), (0, 0)))
    b1r = b1.reshape(1, D1)
    W2p = jnp.pad(W2, ((0, 0), (0, D2 - NCLS)))
    b2r = b2.reshape(1, NCLS)

    degacc = _sc_deg_kernel()(edges)
    y1f, dinv16 = _tc_layer1(x_pad, W1, degacc)
    agg1 = _sc_agg_kernel(D1, 2000, 3)(y1f.reshape(8 * NP, 16), edges)
    y2f = _tc_layer2(agg1, y1f, dinv16, b1r, W2p)
    agg2 = _sc_agg_kernel(D2, 400, 1)(y2f.reshape(2 * NP, 64), edges)
    return _tc_final(agg2, y2f, dinv16, b2r)
